# baseline (speedup 1.0000x reference)
import functools

import jax
import jax.numpy as jnp
from jax import lax
from jax.experimental import pallas as pl
from jax.experimental.pallas import tpu as pltpu
from jax.experimental.pallas import tpu_sc as plsc

_N = 10000
_E = 320000
_D = 128
_H = 64
_EPS = 1e-07

_NS = 16
_B = 80
_NB = _E // _NS // _B
_TILE_EDGES = _NB * _B
_RPT = _N // _NS
_G = 10
_LANES = 16


def _eh_body(x_ref, w_ref, b_ref, o_ref):
    o_ref[...] = (
        jnp.dot(x_ref[...], w_ref[...], preferred_element_type=jnp.float32)
        + b_ref[...]
    )


_BE = 8000


def _edge_encoder(edge_feats, W_edge, b_edge2d):
    return pl.pallas_call(
        _eh_body,
        grid=(_E // _BE,),
        in_specs=[
            pl.BlockSpec((_BE, 16), lambda i: (i, 0)),
            pl.BlockSpec((16, _D), lambda i: (0, 0)),
            pl.BlockSpec((1, _D), lambda i: (0, 0)),
        ],
        out_specs=pl.BlockSpec((_BE, _D), lambda i: (i, 0)),
        out_shape=jax.ShapeDtypeStruct((_E, _D), jnp.float32),
    )(edge_feats, W_edge, b_edge2d)


def _sc_body(node_hbm, eh_hbm, idx_hbm, out_hbm,
             idxv, gbuf, ebuf, acc_sh, lsem, isem, ssem):
    c = lax.axis_index("c")
    s = lax.axis_index("s")
    last = s == _NS - 1

    zero = jnp.zeros((_LANES,), jnp.float32)

    @plsc.parallel_loop(0, _B, unroll=4)
    def _zrow(r):
        for k in range(_D // _LANES):
            gbuf[0, r, pl.ds(k * _LANES, _LANES)] = zero

    r0 = s * _RPT
    for off in range(0, _RPT, _B):
        nr = min(_B, _RPT - off)
        pltpu.sync_copy(gbuf.at[0, pl.ds(0, nr)],
                        acc_sh.at[pl.ds(r0 + off, nr)])

    plsc.subcore_barrier()

    edge0 = s * _TILE_EDGES

    def _load_idx(g, start):
        cp = pltpu.make_async_copy(
            idx_hbm.at[:, pl.ds(s * _NB + g * _G, _G)],
            idxv.at[lax.rem(g, 2)], isem)
        if start:
            cp.start()
        else:
            cp.wait()

    def _idx_ref(j, row):
        g = lax.div(j, _G)
        return idxv.at[lax.rem(g, 2), row, lax.rem(j, _G)]

    def _load_data(j, start):
        p = lax.rem(j, 3)
        g = pltpu.make_async_copy(node_hbm.at[_idx_ref(j, 0)],
                                  gbuf.at[p], lsem)
        e = pltpu.make_async_copy(
            eh_hbm.at[pl.ds(edge0 + j * _B, _B), pl.ds(c * _H, _H)],
            ebuf.at[p], lsem)
        if start:
            g.start()
            e.start()
        else:
            g.wait()
            e.wait()

    def _scatter(j, start):
        p = lax.rem(j, 3)
        cp = pltpu.make_async_copy(gbuf.at[p], acc_sh.at[_idx_ref(j, 1)],
                                   ssem.at[p])
        if start:
            cp.start(add=True)
        else:
            cp.wait()

    def _compute(j):
        p = lax.rem(j, 3)

        def _half(col):
            @plsc.parallel_loop(0, _B, unroll=4)
            def _row(r):
                for k in range(_H // _LANES):
                    ofs = k * _LANES
                    m = (jnp.maximum(gbuf[p, r, pl.ds(col + ofs, _LANES)]
                                     + ebuf[p, r, pl.ds(ofs, _LANES)], 0.0)
                         + _EPS)
                    w = jnp.exp(m)
                    gbuf[p, r, pl.ds(ofs, _LANES)] = m * w
                    gbuf[p, r, pl.ds(_H + ofs, _LANES)] = w

        @pl.when(c == 0)
        def _c0():
            _half(0)

        @pl.when(c == 1)
        def _c1():
            _half(_H)

    _load_idx(0, True)
    _load_idx(0, False)
    _load_idx(1, True)
    _load_data(0, True)

    def _iter(j, _):
        _load_data(j, False)

        @pl.when(j >= 2)
        def _():
            _scatter(j - 2, False)

        @pl.when((lax.rem(j, _G) == 0) & (j >= _G))
        def _():
            _load_idx(lax.div(j, _G), False)

        @pl.when(j + 1 <= _NB - 1)
        def _():
            _load_data(j + 1, True)

        @pl.when((lax.rem(j, _G) == 2) & (j >= _G)
                 & (lax.div(j, _G) + 1 <= _NB // _G - 1))
        def _():
            _load_idx(lax.div(j, _G) + 1, True)

        _compute(j)
        _scatter(j, True)
        return 0

    lax.fori_loop(0, _NB, _iter, 0)
    _scatter(_NB - 2, False)
    _scatter(_NB - 1, False)

    plsc.subcore_barrier()

    def _drain_chunk(row_base, nr):
        pltpu.sync_copy(acc_sh.at[pl.ds(row_base, nr)],
                        gbuf.at[0, pl.ds(0, nr)])

        @plsc.parallel_loop(0, nr, unroll=4)
        def _div(r):
            for k in range(_H // _LANES):
                ofs = k * _LANES
                num = gbuf[0, r, pl.ds(ofs, _LANES)]
                den = gbuf[0, r, pl.ds(_H + ofs, _LANES)]
                ebuf[0, r, pl.ds(ofs, _LANES)] = num / jnp.maximum(den, 1e-16)
        pltpu.sync_copy(ebuf.at[0, pl.ds(0, nr)],
                        out_hbm.at[pl.ds(c * _N + row_base, nr)])

    for off in range(0, _RPT, _B):
        _drain_chunk(r0 + off, min(_B, _RPT - off))


def _sc_aggregate(node_feats, eh_cat, edge_index):
    mesh = plsc.VectorSubcoreMesh(core_axis_name="c", subcore_axis_name="s")
    kern = functools.partial(
        pl.kernel,
        mesh=mesh,
        compiler_params=pltpu.CompilerParams(use_tc_tiling_on_sc=False),
        out_type=jax.ShapeDtypeStruct((2 * _N, _H), jnp.float32),
        scratch_types=[
            pltpu.VMEM((2, 2, _G, _B), jnp.int32),
            pltpu.VMEM((3, _B, _D), jnp.float32),
            pltpu.VMEM((3, _B, _H), jnp.float32),
            pltpu.VMEM_SHARED((_N, _D), jnp.float32),
            pltpu.SemaphoreType.DMA,
            pltpu.SemaphoreType.DMA,
            pltpu.SemaphoreType.DMA((3,)),
        ],
    )(_sc_body)
    return kern(node_feats, eh_cat, edge_index.reshape(2, _E // _B, _B))


def _mlp_body(x_ref, lo_ref, hi_ref, w_ref, b_ref, o_ref):
    feats = x_ref[...] + jnp.concatenate([lo_ref[...], hi_ref[...]], axis=1)
    o_ref[...] = (
        jnp.dot(feats, w_ref[...], preferred_element_type=jnp.float32)
        + b_ref[...]
    )


_BN = 2000


def _output_mlp(node_feats, agg_cat, W_mlp, b_mlp2d):
    nblk = _N // _BN
    return pl.pallas_call(
        _mlp_body,
        grid=(nblk,),
        in_specs=[
            pl.BlockSpec((_BN, _D), lambda i: (i, 0)),
            pl.BlockSpec((_BN, _H), lambda i: (i, 0)),
            pl.BlockSpec((_BN, _H), lambda i: (nblk + i, 0)),
            pl.BlockSpec((_D, _D), lambda i: (0, 0)),
            pl.BlockSpec((1, _D), lambda i: (0, 0)),
        ],
        out_specs=pl.BlockSpec((_BN, _D), lambda i: (i, 0)),
        out_shape=jax.ShapeDtypeStruct((_N, _D), jnp.float32),
    )(node_feats, agg_cat, agg_cat, W_mlp, b_mlp2d)


def kernel(node_feats, edge_feats, edge_index, W_edge, b_edge, W_mlp, b_mlp):
    eh = _edge_encoder(edge_feats, W_edge, b_edge.reshape(1, _D))
    agg_cat = _sc_aggregate(node_feats, eh, edge_index)
    return _output_mlp(node_feats, agg_cat, W_mlp, b_mlp.reshape(1, _D))
